# baseline (speedup 1.0000x reference)
import functools

import jax
import jax.numpy as jnp
from jax import lax
from jax.experimental import pallas as pl
from jax.experimental.pallas import tpu as pltpu
from jax.experimental.pallas import tpu_sc as plsc

DEPTH = 32
B = 4
H = 512
W = 512
NBLK = 8
CHUNK = (H // NBLK) * W
LANES = 16
UNROLL = 8


def _compute_plane(x_v, buf, d):
    def body(j, _):
        base = j * (LANES * UNROLL)
        for u in range(UNROLL):
            off = base + u * LANES
            x = x_v[pl.ds(off, LANES)]
            buf[pl.ds(off, LANES)] = jnp.where(
                x == d, jnp.float32(1.0), jnp.float32(0.0))
        return 0
    lax.fori_loop(0, CHUNK // (LANES * UNROLL), body, 0, unroll=False)


def _sc_body(x_hbm, out_hbm, x_v, buf0, buf1, sem0, sem1):
    nc = 2
    wid = lax.axis_index("s") * nc + lax.axis_index("c")
    b = wid // NBLK
    hblk = wid % NBLK

    pltpu.sync_copy(x_hbm.at[b, hblk], x_v)

    def depth_pair(i, _):
        d0 = 2 * i
        d1 = d0 + 1

        @pl.when(i > 0)
        def _():
            pltpu.make_async_copy(buf0, out_hbm.at[b, d0, hblk], sem0).wait()

        _compute_plane(x_v, buf0, d0)
        pltpu.make_async_copy(buf0, out_hbm.at[b, d0, hblk], sem0).start()

        @pl.when(i > 0)
        def _():
            pltpu.make_async_copy(buf1, out_hbm.at[b, d1, hblk], sem1).wait()

        _compute_plane(x_v, buf1, d1)
        pltpu.make_async_copy(buf1, out_hbm.at[b, d1, hblk], sem1).start()
        return 0

    lax.fori_loop(0, DEPTH // 2, depth_pair, 0, unroll=False)

    pltpu.make_async_copy(buf0, out_hbm.at[b, 0, hblk], sem0).wait()
    pltpu.make_async_copy(buf1, out_hbm.at[b, 1, hblk], sem1).wait()


def kernel(rank, X_in, ones):
    x = X_in.reshape(B, NBLK, CHUNK)
    mesh = plsc.VectorSubcoreMesh(core_axis_name="c", subcore_axis_name="s")
    run = functools.partial(
        pl.kernel,
        mesh=mesh,
        out_type=jax.ShapeDtypeStruct((B, DEPTH, NBLK, CHUNK), jnp.float32),
        scratch_types=[
            pltpu.VMEM((CHUNK,), jnp.int32),
            pltpu.VMEM((CHUNK,), jnp.float32),
            pltpu.VMEM((CHUNK,), jnp.float32),
            pltpu.SemaphoreType.DMA,
            pltpu.SemaphoreType.DMA,
        ],
    )(_sc_body)
    out = run(x)
    return out.reshape(B, DEPTH, H, W)
